# Initial kernel scaffold; baseline (speedup 1.0000x reference)
import jax
import jax.numpy as jnp
from jax.experimental import pallas as pl


def kernel(x, edge_index, W1_l, W1_r, b1, W2_l, W2_r, b2):
    raise NotImplementedError("write your pallas kernel here")


# speedup vs baseline: 12.3652x; 12.3652x over previous
import functools

import jax
import jax.numpy as jnp
from jax import lax
from jax.experimental import pallas as pl
from jax.experimental.pallas import tpu as pltpu
from jax.experimental.pallas import tpu_sc as plsc

N = 10000
E = 320000
DF = 128
DH = 256
DA = DF + 16

NC = 2
NS = 16
NW = NC * NS
EW = E // NW
C = 80
S = 25
NSC = EW // (S * C)
RT = N // NS


def _make_sc_agg(D):
  mesh = plsc.VectorSubcoreMesh(core_axis_name="c", subcore_axis_name="s")

  @functools.partial(
      pl.kernel,
      out_type=jax.ShapeDtypeStruct((NC, N, D), jnp.float32),
      mesh=mesh,
      compiler_params=pltpu.CompilerParams(use_tc_tiling_on_sc=False),
      scratch_types=[
          pltpu.VMEM((S, C), jnp.int32),
          pltpu.VMEM((S, C), jnp.int32),
          pltpu.VMEM((C, D), jnp.float32),
          pltpu.VMEM((C, D), jnp.float32),
          pltpu.VMEM_SHARED((N, D), jnp.float32),
          pltpu.SemaphoreType.DMA,
          pltpu.SemaphoreType.DMA,
      ],
  )
  def sc_agg(table, srcr, dstr, zrow, out, isrc, idst, rows0, rows1, acc,
             sem0, sem1):
    c = lax.axis_index("c")
    s = lax.axis_index("s")
    w = s * NC + c
    base = s * RT

    pltpu.sync_copy(zrow, acc.at[pl.ds(base, RT)])
    plsc.subcore_barrier()

    @pl.loop(0, NSC)
    def _super(u):
      pltpu.sync_copy(srcr.at[w].at[u], isrc)
      pltpu.sync_copy(dstr.at[w].at[u], idst)
      pltpu.async_copy(table.at[isrc.at[0]], rows0, sem0)

      @pl.loop(0, S - 1, step=2)
      def _edges(k):
        pltpu.async_copy(table.at[isrc.at[k + 1]], rows1, sem1)
        pltpu.make_async_copy(table.at[isrc.at[k]], rows0, sem0).wait()
        pltpu.sync_copy(rows0, acc.at[idst.at[k]], add=True)

        @pl.when(k + 2 < S)
        def _():
          pltpu.async_copy(table.at[isrc.at[k + 2]], rows0, sem0)

        pltpu.make_async_copy(table.at[isrc.at[k + 1]], rows1, sem1).wait()
        pltpu.sync_copy(rows1, acc.at[idst.at[k + 1]], add=True)

      pltpu.make_async_copy(table.at[isrc.at[S - 1]], rows0, sem0).wait()
      pltpu.sync_copy(rows0, acc.at[idst.at[S - 1]], add=True)

    plsc.subcore_barrier()

    pltpu.sync_copy(acc.at[pl.ds(base, RT)], out.at[c].at[pl.ds(base, RT)])

  return sc_agg


_sc_agg_a = _make_sc_agg(DA)
_sc_agg_f = _make_sc_agg(DF)

BN = 1000
GRID = N // BN


def _tc1_body(aggc, x, w1l, w1r, b1, w2l, w2r, b2, g, hr, inv):
  ac = aggc[0] + aggc[1]
  agg = ac[:, :DF]
  cnt = ac[:, DF:DF + 1]
  iv = 1.0 / jnp.maximum(cnt, 1.0)
  mean = agg * iv
  h = (jnp.dot(mean, w1l[...], preferred_element_type=jnp.float32)
       + jnp.dot(x[...], w1r[...], preferred_element_type=jnp.float32)
       + b1[...])
  h = jnp.maximum(h, 0.0)
  g[...] = jnp.dot(h, w2l[...], preferred_element_type=jnp.float32)
  hr[...] = (jnp.dot(h, w2r[...], preferred_element_type=jnp.float32)
             + b2[...])
  inv[...] = iv


_tc1 = pl.pallas_call(
    _tc1_body,
    grid=(GRID,),
    in_specs=[
        pl.BlockSpec((NC, BN, DA), lambda i: (0, i, 0)),
        pl.BlockSpec((BN, DF), lambda i: (i, 0)),
        pl.BlockSpec((DF, DH), lambda i: (0, 0)),
        pl.BlockSpec((DF, DH), lambda i: (0, 0)),
        pl.BlockSpec((1, DH), lambda i: (0, 0)),
        pl.BlockSpec((DH, DF), lambda i: (0, 0)),
        pl.BlockSpec((DH, DF), lambda i: (0, 0)),
        pl.BlockSpec((1, DF), lambda i: (0, 0)),
    ],
    out_specs=[
        pl.BlockSpec((BN, DF), lambda i: (i, 0)),
        pl.BlockSpec((BN, DF), lambda i: (i, 0)),
        pl.BlockSpec((BN, 1), lambda i: (i, 0)),
    ],
    out_shape=[
        jax.ShapeDtypeStruct((N, DF), jnp.float32),
        jax.ShapeDtypeStruct((N, DF), jnp.float32),
        jax.ShapeDtypeStruct((N, 1), jnp.float32),
    ],
)


def _tc2_body(agg2, inv, hr, out):
  out[...] = (agg2[0] + agg2[1]) * inv[...] + hr[...]


_tc2 = pl.pallas_call(
    _tc2_body,
    grid=(GRID,),
    in_specs=[
        pl.BlockSpec((NC, BN, DF), lambda i: (0, i, 0)),
        pl.BlockSpec((BN, 1), lambda i: (i, 0)),
        pl.BlockSpec((BN, DF), lambda i: (i, 0)),
    ],
    out_specs=pl.BlockSpec((BN, DF), lambda i: (i, 0)),
    out_shape=jax.ShapeDtypeStruct((N, DF), jnp.float32),
)


def kernel(x, edge_index, W1_l, W1_r, b1, W2_l, W2_r, b2):
  src = edge_index[0].astype(jnp.int32).reshape(NW, NSC, S, C)
  dst = edge_index[1].astype(jnp.int32).reshape(NW, NSC, S, C)
  xa = jnp.concatenate([x, jnp.ones((N, DA - DF), jnp.float32)], axis=1)
  zrow_a = jnp.zeros((RT, DA), jnp.float32)
  zrow_f = jnp.zeros((RT, DF), jnp.float32)

  aggc = _sc_agg_a(xa, src, dst, zrow_a)
  g, hr, inv = _tc1(aggc, x, W1_l, W1_r, b1.reshape(1, DH),
                    W2_l, W2_r, b2.reshape(1, DF))
  agg2 = _sc_agg_f(g, src, dst, zrow_f)
  return _tc2(agg2, inv, hr)
